# baseline (speedup 1.0000x reference)
import functools

import jax
import jax.numpy as jnp
from jax import lax
from jax.experimental import pallas as pl
from jax.experimental.pallas import tpu as pltpu
from jax.experimental.pallas import tpu_sc as plsc

NC = 2
NS = 16
NW = NC * NS
CH = 128


def _cdiv(a, b):
    return (a + b - 1) // b


def _make_deg(cpw_deg, n_pad, rpw):
    mesh = plsc.VectorSubcoreMesh(core_axis_name="c", subcore_axis_name="s",
                                  num_cores=NC, num_subcores=NS)

    @functools.partial(
        pl.kernel,
        out_type=jax.ShapeDtypeStruct((NC * n_pad,), jnp.float32),
        mesh=mesh,
        compiler_params=pltpu.CompilerParams(needs_layout_passes=False),
        scratch_types=[
            pltpu.VMEM((8 * CH,), jnp.int32),
            pltpu.VMEM((n_pad,), jnp.float32),
            pltpu.VMEM((rpw,), jnp.float32),
            pltpu.VMEM((rpw,), jnp.float32),
            pltpu.VMEM_SHARED((NS * n_pad,), jnp.float32),
        ],
    )
    def deg_kernel(dst_hbm, out_hbm, idx_v, hist, accv, tbuf, shists):
        c = lax.axis_index("c")
        s = lax.axis_index("s")
        w = c * NS + s
        base = w * (cpw_deg * CH)
        zeros16 = jnp.zeros((16,), jnp.float32)
        ones16 = jnp.ones((16,), jnp.float32)

        def zrow(i, _):
            hist[pl.ds(i * 16, 16)] = zeros16
            return 0

        lax.fori_loop(0, n_pad // 16, zrow, 0)

        def chunk(i, _):
            pltpu.sync_copy(dst_hbm.at[pl.ds(base + i * 8 * CH, 8 * CH)],
                            idx_v)

            def step(j, _):
                plsc.addupdate_scatter(hist, [idx_v[pl.ds(j * 16, 16)]],
                                       ones16)
                return 0

            lax.fori_loop(0, 8 * CH // 16, step, 0)
            return 0

        lax.fori_loop(0, cpw_deg // 8, chunk, 0)

        def chunk_rem(i, _):
            pltpu.sync_copy(
                dst_hbm.at[pl.ds(base + (cpw_deg // 8) * 8 * CH + i * CH,
                                 CH)],
                idx_v.at[pl.ds(0, CH)])

            def step(j, _):
                plsc.addupdate_scatter(hist, [idx_v[pl.ds(j * 16, 16)]],
                                       ones16)
                return 0

            lax.fori_loop(0, CH // 16, step, 0)
            return 0

        lax.fori_loop(0, cpw_deg % 8, chunk_rem, 0)

        pltpu.sync_copy(hist, shists.at[pl.ds(s * n_pad, n_pad)])
        plsc.subcore_barrier()
        pltpu.sync_copy(shists.at[pl.ds(s * rpw, rpw)], accv)
        for t in range(1, NS):
            pltpu.sync_copy(shists.at[pl.ds(t * n_pad + s * rpw, rpw)], tbuf)

            def addrow(i, _):
                accv[pl.ds(i * 16, 16)] = (accv[pl.ds(i * 16, 16)]
                                           + tbuf[pl.ds(i * 16, 16)])
                return 0

            lax.fori_loop(0, rpw // 16, addrow, 0)
        pltpu.sync_copy(accv, out_hbm.at[pl.ds(c * n_pad + s * rpw, rpw)])

    return deg_kernel


def _make_agg(cpw, dw, n_pad):
    mesh = plsc.VectorSubcoreMesh(core_axis_name="c", subcore_axis_name="s",
                                  num_cores=NC, num_subcores=NS)
    rpw = n_pad // NS

    @functools.partial(
        pl.kernel,
        out_type=jax.ShapeDtypeStruct((n_pad, dw), jnp.float32),
        mesh=mesh,
        scratch_types=[
            pltpu.VMEM((CH,), jnp.int32),
            pltpu.VMEM((CH,), jnp.int32),
            pltpu.VMEM((CH,), jnp.int32),
            pltpu.VMEM((CH,), jnp.int32),
            pltpu.VMEM((CH,), jnp.int32),
            pltpu.VMEM((CH,), jnp.int32),
            pltpu.VMEM((CH, dw), jnp.float32),
            pltpu.VMEM((CH, dw), jnp.float32),
            pltpu.VMEM((CH, dw), jnp.float32),
            pltpu.VMEM_SHARED((n_pad, dw), jnp.float32),
            pltpu.SemaphoreType.DMA,
            pltpu.SemaphoreType.DMA,
            pltpu.SemaphoreType.DMA,
            pltpu.SemaphoreType.DMA,
            pltpu.SemaphoreType.DMA,
            pltpu.SemaphoreType.DMA,
        ],
    )
    def agg_kernel(src_hbm, dst_hbm, g_hbm, out_hbm,
                   s0, s1, s2, d0, d1, d2, r0, r1, r2, acc,
                   gm0, gm1, gm2, sm0, sm1, sm2):
        c = lax.axis_index("c")
        s = lax.axis_index("s")

        @pl.when(c == 0)
        def _core0():
            base = s * (cpw * CH)
            srcs, dsts, rows = (s0, s1, s2), (d0, d1, d2), (r0, r1, r2)
            gsems, ssems = (gm0, gm1, gm2), (sm0, sm1, sm2)

            def fillz(i, _):
                for j in range(dw // 16):
                    r0[i, pl.ds(j * 16, 16)] = jnp.zeros((16,), jnp.float32)
                return 0

            lax.fori_loop(0, CH, fillz, 0)
            for k in range(rpw // CH):
                pltpu.sync_copy(r0, acc.at[pl.ds(s * rpw + k * CH, CH)])
            rem = rpw % CH
            if rem:
                pltpu.sync_copy(r0.at[pl.ds(0, rem)],
                                acc.at[pl.ds(s * rpw + (rpw // CH) * CH,
                                             rem)])
            plsc.subcore_barrier()

            def load(b, chunk):
                off = base + chunk * CH
                pltpu.sync_copy(src_hbm.at[pl.ds(off, CH)], srcs[b])
                pltpu.sync_copy(dst_hbm.at[pl.ds(off, CH)], dsts[b])
                pltpu.async_copy(g_hbm.at[srcs[b]], rows[b], gsems[b])

            def wait_g(b):
                pltpu.make_async_copy(g_hbm.at[srcs[b]], rows[b],
                                      gsems[b]).wait()

            def scat(b):
                pltpu.async_copy(rows[b], acc.at[dsts[b]], ssems[b],
                                 add=True)

            def wait_s(b):
                pltpu.make_async_copy(rows[b], acc.at[dsts[b]],
                                      ssems[b]).wait()

            for b in range(3):
                load(b, b)

            def body(k, _):
                for b in range(3):
                    i = 3 * k + b
                    wait_g(b)
                    scat(b)
                    pltpu.sync_copy(
                        src_hbm.at[pl.ds(base + (i + 3) * CH, CH)], srcs[b])
                    wait_s(b)
                    pltpu.sync_copy(
                        dst_hbm.at[pl.ds(base + (i + 3) * CH, CH)], dsts[b])
                    pltpu.async_copy(g_hbm.at[srcs[b]], rows[b], gsems[b])
                return 0

            lax.fori_loop(0, cpw // 3 - 1, body, 0)
            for b in range(3):
                wait_g(b)
                scat(b)
                wait_s(b)
            plsc.subcore_barrier()
            pltpu.sync_copy(acc.at[pl.ds(s * rpw, rpw)],
                            out_hbm.at[pl.ds(s * rpw, rpw)])

    return agg_kernel


def _tc_stage1(n, n_pad):
    def body(x_ref, gam_ref, bet_ref, w1_ref, dp_ref, g1_ref, dis_ref):
        x = x_ref[...]
        mean = jnp.mean(x, axis=0, keepdims=True)
        xc = x - mean
        var = jnp.mean(xc * xc, axis=0, keepdims=True)
        xh = xc * lax.rsqrt(var + 1e-5) * gam_ref[...] + bet_ref[...]
        deg = (dp_ref[:n_pad] + dp_ref[n_pad:]).reshape(n_pad, 1) + 1.0
        dis = lax.rsqrt(deg)
        dis_ref[...] = dis
        h1 = jnp.dot(xh, w1_ref[...], preferred_element_type=jnp.float32)
        g1_ref[...] = h1 * dis[:n]

    return body


def _tc_stage2(n):
    def body(s1_ref, g1_ref, dis_ref, b1_ref, w2_ref, g2_ref):
        dis = dis_ref[...][:n]
        ssum = s1_ref[:n, :] + g1_ref[...]
        o1 = jnp.maximum(ssum * dis + b1_ref[...], 0.0)
        h2 = jnp.dot(o1, w2_ref[...], preferred_element_type=jnp.float32)
        g2_ref[...] = h2 * dis

    return body


def _tc_stage3(n, c_out):
    def body(s2_ref, g2_ref, dis_ref, b2_ref, out_ref):
        dis = dis_ref[...][:n]
        ssum = s2_ref[:n, :] + g2_ref[...]
        o = (ssum * dis)[:, :c_out] + b2_ref[...]
        m = jnp.max(o, axis=1, keepdims=True)
        e = jnp.exp(o - m)
        lse = jnp.log(jnp.sum(e, axis=1, keepdims=True)) + m
        out_ref[...] = o - lse

    return body


def kernel(x, edge_index, bn_gamma, bn_beta, W1, b1, W2, b2):
    n, d = x.shape
    h = W1.shape[1]
    c_out = W2.shape[1]
    e = edge_index.shape[1]

    cpw = _cdiv(_cdiv(e, NS * CH), 3) * 3
    cpw_deg = _cdiv(e, NW * CH)
    e_pad = max(NS * CH * cpw, NW * CH * cpw_deg)
    cpw_deg = e_pad // (NW * CH)
    rpw = _cdiv(n + 1, NS * 8) * 8
    n_pad = rpw * NS
    rpw_deg = _cdiv(n + 1, NS * 16) * 16
    n_pad_deg = rpw_deg * NS

    src = edge_index[0]
    dst = edge_index[1]
    pad = e_pad - e
    src_p = jnp.concatenate([src, jnp.zeros((pad,), jnp.int32)])
    dst_p = jnp.concatenate([dst, jnp.full((pad,), n, jnp.int32)])

    dw2 = _cdiv(c_out, 128) * 128
    w2_p = jnp.pad(W2, ((0, 0), (0, dw2 - c_out)))

    deg_partial = _make_deg(cpw_deg, n_pad_deg, rpw_deg)(dst_p)

    g1, dis = pl.pallas_call(
        _tc_stage1(n, n_pad_deg),
        out_shape=(jax.ShapeDtypeStruct((n, h), jnp.float32),
                   jax.ShapeDtypeStruct((n_pad_deg, 1), jnp.float32)),
    )(x, bn_gamma, bn_beta, W1, deg_partial)

    s1 = _make_agg(cpw, h, n_pad)(src_p, dst_p, g1)

    g2 = pl.pallas_call(
        _tc_stage2(n),
        out_shape=jax.ShapeDtypeStruct((n, dw2), jnp.float32),
    )(s1, g1, dis, b1, w2_p)

    s2 = _make_agg(cpw, dw2, n_pad)(src_p, dst_p, g2)

    out = pl.pallas_call(
        _tc_stage3(n, c_out),
        out_shape=jax.ShapeDtypeStruct((n, c_out), jnp.float32),
    )(s2, g2, dis, b2)

    return out


# speedup vs baseline: 1.1845x; 1.1845x over previous
import functools

import jax
import jax.numpy as jnp
from jax import lax
from jax.experimental import pallas as pl
from jax.experimental.pallas import tpu as pltpu
from jax.experimental.pallas import tpu_sc as plsc

NC = 2
NS = 16
NW = NC * NS
CH = 128


def _cdiv(a, b):
    return (a + b - 1) // b


def _make_deg(cpw_deg, n_pad, rpw):
    mesh = plsc.VectorSubcoreMesh(core_axis_name="c", subcore_axis_name="s",
                                  num_cores=NC, num_subcores=NS)

    @functools.partial(
        pl.kernel,
        out_type=jax.ShapeDtypeStruct((NC * n_pad,), jnp.float32),
        mesh=mesh,
        compiler_params=pltpu.CompilerParams(needs_layout_passes=False),
        scratch_types=[
            pltpu.VMEM((8 * CH,), jnp.int32),
            pltpu.VMEM((n_pad,), jnp.float32),
            pltpu.VMEM((rpw,), jnp.float32),
            pltpu.VMEM((rpw,), jnp.float32),
            pltpu.VMEM_SHARED((NS * n_pad,), jnp.float32),
        ],
    )
    def deg_kernel(dst_hbm, out_hbm, idx_v, hist, accv, tbuf, shists):
        c = lax.axis_index("c")
        s = lax.axis_index("s")
        w = c * NS + s
        base = w * (cpw_deg * CH)
        zeros16 = jnp.zeros((16,), jnp.float32)
        ones16 = jnp.ones((16,), jnp.float32)

        def zrow(i, _):
            hist[pl.ds(i * 16, 16)] = zeros16
            return 0

        lax.fori_loop(0, n_pad // 16, zrow, 0)

        def chunk(i, _):
            pltpu.sync_copy(dst_hbm.at[pl.ds(base + i * 8 * CH, 8 * CH)],
                            idx_v)

            def step(j, _):
                plsc.addupdate_scatter(hist, [idx_v[pl.ds(j * 16, 16)]],
                                       ones16)
                return 0

            lax.fori_loop(0, 8 * CH // 16, step, 0)
            return 0

        lax.fori_loop(0, cpw_deg // 8, chunk, 0)

        def chunk_rem(i, _):
            pltpu.sync_copy(
                dst_hbm.at[pl.ds(base + (cpw_deg // 8) * 8 * CH + i * CH,
                                 CH)],
                idx_v.at[pl.ds(0, CH)])

            def step(j, _):
                plsc.addupdate_scatter(hist, [idx_v[pl.ds(j * 16, 16)]],
                                       ones16)
                return 0

            lax.fori_loop(0, CH // 16, step, 0)
            return 0

        lax.fori_loop(0, cpw_deg % 8, chunk_rem, 0)

        pltpu.sync_copy(hist, shists.at[pl.ds(s * n_pad, n_pad)])
        plsc.subcore_barrier()
        pltpu.sync_copy(shists.at[pl.ds(s * rpw, rpw)], accv)
        for t in range(1, NS):
            pltpu.sync_copy(shists.at[pl.ds(t * n_pad + s * rpw, rpw)], tbuf)

            def addrow(i, _):
                accv[pl.ds(i * 16, 16)] = (accv[pl.ds(i * 16, 16)]
                                           + tbuf[pl.ds(i * 16, 16)])
                return 0

            lax.fori_loop(0, rpw // 16, addrow, 0)
        pltpu.sync_copy(accv, out_hbm.at[pl.ds(c * n_pad + s * rpw, rpw)])

    return deg_kernel


def _make_agg(cpw, dw, n_pad):
    mesh = plsc.VectorSubcoreMesh(core_axis_name="c", subcore_axis_name="s",
                                  num_cores=NC, num_subcores=NS)
    rpw = n_pad // NS

    @functools.partial(
        pl.kernel,
        out_type=jax.ShapeDtypeStruct((n_pad, dw), jnp.float32),
        mesh=mesh,
        scratch_types=[
            pltpu.VMEM((CH,), jnp.int32),
            pltpu.VMEM((CH,), jnp.int32),
            pltpu.VMEM((CH,), jnp.int32),
            pltpu.VMEM((CH,), jnp.int32),
            pltpu.VMEM((CH, dw), jnp.float32),
            pltpu.VMEM((CH, dw), jnp.float32),
            pltpu.VMEM_SHARED((n_pad, dw), jnp.float32),
            pltpu.SemaphoreType.DMA,
            pltpu.SemaphoreType.DMA,
            pltpu.SemaphoreType.DMA,
            pltpu.SemaphoreType.DMA,
        ],
    )
    def agg_kernel(src_hbm, dst_hbm, g_hbm, out_hbm,
                   sa, sb, da, db, ra, rb, acc, sem_a, sem_b, ssem_a, ssem_b):
        c = lax.axis_index("c")
        s = lax.axis_index("s")

        @pl.when(c == 0)
        def _core0():
            base = s * (cpw * CH)
            srcs, dsts, rows = (sa, sb), (da, db), (ra, rb)
            sems, ssems = (sem_a, sem_b), (ssem_a, ssem_b)

            def fillz(i, _):
                for j in range(dw // 16):
                    ra[i, pl.ds(j * 16, 16)] = jnp.zeros((16,), jnp.float32)
                return 0

            lax.fori_loop(0, CH, fillz, 0)
            for k in range(rpw // CH):
                pltpu.sync_copy(ra, acc.at[pl.ds(s * rpw + k * CH, CH)])
            rem = rpw % CH
            if rem:
                pltpu.sync_copy(ra.at[pl.ds(0, rem)],
                                acc.at[pl.ds(s * rpw + (rpw // CH) * CH,
                                             rem)])
            plsc.subcore_barrier()

            def load(b, chunk):
                off = base + chunk * CH
                pltpu.sync_copy(src_hbm.at[pl.ds(off, CH)], srcs[b])
                pltpu.sync_copy(dst_hbm.at[pl.ds(off, CH)], dsts[b])
                pltpu.async_copy(g_hbm.at[srcs[b]], rows[b], sems[b])

            def wait_g(b):
                pltpu.make_async_copy(g_hbm.at[srcs[b]], rows[b],
                                      sems[b]).wait()

            def scat(b):
                pltpu.async_copy(rows[b], acc.at[dsts[b]], ssems[b],
                                 add=True)

            def wait_s(b):
                pltpu.make_async_copy(rows[b], acc.at[dsts[b]],
                                      ssems[b]).wait()

            load(0, 0)
            load(1, 1)

            def body(k, _):
                for b in (0, 1):
                    i = 2 * k + b
                    wait_g(b)
                    scat(b)
                    pltpu.sync_copy(
                        src_hbm.at[pl.ds(base + (i + 2) * CH, CH)], srcs[b])
                    wait_s(b)
                    pltpu.sync_copy(
                        dst_hbm.at[pl.ds(base + (i + 2) * CH, CH)], dsts[b])
                    pltpu.async_copy(g_hbm.at[srcs[b]], rows[b], sems[b])
                return 0

            lax.fori_loop(0, (cpw - 2) // 2, body, 0)
            for b in (0, 1):
                wait_g(b)
                scat(b)
                wait_s(b)
            plsc.subcore_barrier()
            pltpu.sync_copy(acc.at[pl.ds(s * rpw, rpw)],
                            out_hbm.at[pl.ds(s * rpw, rpw)])

    return agg_kernel


def _tc_stage1(n, n_pad):
    def body(x_ref, gam_ref, bet_ref, w1_ref, dp_ref, g1_ref, dis_ref):
        x = x_ref[...]
        mean = jnp.mean(x, axis=0, keepdims=True)
        xc = x - mean
        var = jnp.mean(xc * xc, axis=0, keepdims=True)
        xh = xc * lax.rsqrt(var + 1e-5) * gam_ref[...] + bet_ref[...]
        deg = (dp_ref[:n_pad] + dp_ref[n_pad:]).reshape(n_pad, 1) + 1.0
        dis = lax.rsqrt(deg)
        dis_ref[...] = dis
        h1 = jnp.dot(xh, w1_ref[...], preferred_element_type=jnp.float32)
        g1_ref[...] = h1 * dis[:n]

    return body


def _tc_stage2(n):
    def body(s1_ref, g1_ref, dis_ref, b1_ref, w2_ref, g2_ref):
        dis = dis_ref[...][:n]
        ssum = s1_ref[:n, :] + g1_ref[...]
        o1 = jnp.maximum(ssum * dis + b1_ref[...], 0.0)
        h2 = jnp.dot(o1, w2_ref[...], preferred_element_type=jnp.float32)
        g2_ref[...] = h2 * dis

    return body


def _tc_stage3(n, c_out):
    def body(s2_ref, g2_ref, dis_ref, b2_ref, out_ref):
        dis = dis_ref[...][:n]
        ssum = s2_ref[:n, :] + g2_ref[...]
        o = (ssum * dis)[:, :c_out] + b2_ref[...]
        m = jnp.max(o, axis=1, keepdims=True)
        e = jnp.exp(o - m)
        lse = jnp.log(jnp.sum(e, axis=1, keepdims=True)) + m
        out_ref[...] = o - lse

    return body


def kernel(x, edge_index, bn_gamma, bn_beta, W1, b1, W2, b2):
    n, d = x.shape
    h = W1.shape[1]
    c_out = W2.shape[1]
    e = edge_index.shape[1]

    cpw = _cdiv(e, NS * CH)
    cpw += cpw % 2
    e_pad = NS * CH * cpw
    cpw_deg = cpw // 2
    rpw = _cdiv(n + 1, NS * 16) * 16
    n_pad = rpw * NS

    src = edge_index[0]
    dst = edge_index[1]
    pad = e_pad - e
    src_p = jnp.concatenate([src, jnp.zeros((pad,), jnp.int32)])
    dst_p = jnp.concatenate([dst, jnp.full((pad,), n, jnp.int32)])

    dw2 = _cdiv(c_out, 128) * 128
    w2_p = jnp.pad(W2, ((0, 0), (0, dw2 - c_out)))

    deg_partial = _make_deg(cpw_deg, n_pad, rpw)(dst_p)

    g1, dis = pl.pallas_call(
        _tc_stage1(n, n_pad),
        out_shape=(jax.ShapeDtypeStruct((n, h), jnp.float32),
                   jax.ShapeDtypeStruct((n_pad, 1), jnp.float32)),
    )(x, bn_gamma, bn_beta, W1, deg_partial)

    s1 = _make_agg(cpw, h, n_pad)(src_p, dst_p, g1)

    g2 = pl.pallas_call(
        _tc_stage2(n),
        out_shape=jax.ShapeDtypeStruct((n, dw2), jnp.float32),
    )(s1, g1, dis, b1, w2_p)

    s2 = _make_agg(cpw, dw2, n_pad)(src_p, dst_p, g2)

    out = pl.pallas_call(
        _tc_stage3(n, c_out),
        out_shape=jax.ShapeDtypeStruct((n, c_out), jnp.float32),
    )(s2, g2, dis, b2)

    return out
